# Initial kernel scaffold; baseline (speedup 1.0000x reference)
import jax
import jax.numpy as jnp
from jax.experimental import pallas as pl


def kernel(volume, conv_kernel, conv_bias):
    raise NotImplementedError("write your pallas kernel here")


# speedup vs baseline: 253.9504x; 253.9504x over previous
import functools

import jax
import jax.numpy as jnp
from jax import lax
from jax.experimental import pallas as pl
from jax.experimental.pallas import tpu as pltpu
from jax.experimental.pallas import tpu_sc as plsc

B, H, W, C = 2, 96, 96, 64
OH, OW = 80, 88
NTAP, NOFF = 25, 100
G = 2
N_ELEM = B * OH * OW * 2 * NTAP
NC, NS = 2, 16
NW = NC * NS
PER_W = N_ELEM // NW
VECS = PER_W // 16


def _conv_body(vol_ref, w_ref, b_ref, off_ref, s_ref):
    r = pl.program_id(1)
    acc = jnp.zeros((OW, NOFF), jnp.float32)
    for t in range(NTAP):
        i, j = t // 5, t % 5
        a = vol_ref[0, r + 4 * i, pl.ds(2 * j, OW), :]
        acc += jnp.dot(a, w_ref[t], preferred_element_type=jnp.float32)
    off_ref[0, 0] = acc + b_ref[0]

    @pl.when(r == 0)
    def _():
        s_ref[0] = jnp.sum(vol_ref[0], axis=-1)


def _conv_call(vol, wp, bp):
    return pl.pallas_call(
        _conv_body,
        grid=(B, OH),
        in_specs=[
            pl.BlockSpec((1, H, W, C), lambda b, r: (b, 0, 0, 0)),
            pl.BlockSpec((NTAP, C, NOFF), lambda b, r: (0, 0, 0)),
            pl.BlockSpec((1, NOFF), lambda b, r: (0, 0)),
        ],
        out_specs=[
            pl.BlockSpec((1, 1, OW, NOFF), lambda b, r: (b, r, 0, 0)),
            pl.BlockSpec((1, H, W), lambda b, r: (b, 0, 0)),
        ],
        out_shape=[
            jax.ShapeDtypeStruct((B, OH, OW, NOFF), jnp.float32),
            jax.ShapeDtypeStruct((B, H, W), jnp.float32),
        ],
    )(vol, wp, bp)


def _sc_body(dy_hbm, dx_hbm, s_hbm, out_hbm, dy_v, dx_v, s_v, out_v):
    cid = lax.axis_index("c")
    sid = lax.axis_index("s")
    wid = sid * NC + cid
    b = wid // NS
    base = wid * PER_W
    pltpu.sync_copy(s_hbm.at[b], s_v)
    pltpu.sync_copy(dy_hbm.at[pl.ds(base, PER_W)], dy_v)
    pltpu.sync_copy(dx_hbm.at[pl.ds(base, PER_W)], dx_v)

    def step(v, carry):
        off = v * 16
        gidx = base + off + lax.iota(jnp.int32, 16)
        dyv = dy_v[pl.ds(off, 16)]
        dxv = dx_v[pl.ds(off, 16)]
        k = lax.rem(gidx, NTAP)
        ki = lax.div(k, 5)
        kj = k - 5 * ki
        ky = (ki * 4 - 8).astype(jnp.float32)
        kx = (kj * 2 - 4).astype(jnp.float32)
        py = lax.rem(lax.div(gidx, OW * 2 * NTAP), OH)
        yf = (py + 8).astype(jnp.float32)
        rx = ky + dyv
        ry = kx + dxv
        x0 = rx.astype(jnp.int32)
        y0 = ry.astype(jnp.int32)
        x1 = x0 + 1
        y1 = y0 + 1
        y0c = jnp.clip(y0, 0, H - 1)
        y1c = jnp.clip(y1, 0, H - 1)
        x0c = jnp.clip(x0, 0, W - 1)
        x1c = jnp.clip(x1, 0, W - 1)
        p0 = plsc.load_gather(s_v, [y0c, x0c])
        p1 = plsc.load_gather(s_v, [y0c, x1c])
        p2 = plsc.load_gather(s_v, [y1c, x0c])
        p3 = plsc.load_gather(s_v, [y1c, x1c])
        y0f = y0c.astype(jnp.float32)
        y1f = y1c.astype(jnp.float32)
        x0f = x0c.astype(jnp.float32)
        x1f = x1c.astype(jnp.float32)
        w0 = (y1f - ry) * (x1f - rx)
        w1 = (y1f - yf) * (rx - x0f)
        w2 = (ry - y0f) * (x1f - rx)
        w3 = (ry - y0f) * (rx - x0f)
        out_v[pl.ds(off, 16)] = p0 * w0 + p1 * w1 + p2 * w2 + p3 * w3
        return carry

    lax.fori_loop(0, VECS, step, 0)
    pltpu.sync_copy(out_v, out_hbm.at[pl.ds(base, PER_W)])


def _sc_call(dy, dx, s):
    mesh = plsc.VectorSubcoreMesh(core_axis_name="c", subcore_axis_name="s",
                                  num_cores=NC, num_subcores=NS)
    f = pl.kernel(
        _sc_body,
        out_type=jax.ShapeDtypeStruct((N_ELEM,), jnp.float32),
        mesh=mesh,
        compiler_params=pltpu.CompilerParams(needs_layout_passes=False),
        scratch_types=[
            pltpu.VMEM((PER_W,), jnp.float32),
            pltpu.VMEM((PER_W,), jnp.float32),
            pltpu.VMEM((H, W), jnp.float32),
            pltpu.VMEM((PER_W,), jnp.float32),
        ],
    )
    return f(dy, dx, s)


@jax.jit
def kernel(volume, conv_kernel, conv_bias):
    wp = (conv_kernel.reshape(5, 5, C, NTAP, 2, G)
          .transpose(0, 1, 2, 4, 5, 3)
          .reshape(5, 5, C, NOFF)
          .reshape(NTAP, C, NOFF))
    bp = (conv_bias.reshape(NTAP, 2, G).transpose(1, 2, 0)
          .reshape(1, NOFF))
    off, s = _conv_call(volume, wp, bp)
    dy = off[..., :50].reshape(-1)
    dx = off[..., 50:].reshape(-1)
    out_flat = _sc_call(dy, dx, s)
    return (out_flat.reshape(B, OH, OW, G, NTAP)
            .transpose(0, 3, 1, 2, 4)
            .reshape(B, G, OH, OW, 5, 5))
